# Initial kernel scaffold; baseline (speedup 1.0000x reference)
import jax
import jax.numpy as jnp
from jax.experimental import pallas as pl


def kernel(x, segment_ids, token_table, pos_table, seg_table, ln_gamma, ln_beta):
    raise NotImplementedError("write your pallas kernel here")


# speedup vs baseline: 1.9742x; 1.9742x over previous
import functools

import jax
import jax.numpy as jnp
from jax import lax
from jax.experimental import pallas as pl
from jax.experimental.pallas import tpu as pltpu
from jax.experimental.pallas import tpu_sc as plsc

B = 1024
S = 200
E = 64
R = B * S

_info = plsc.get_sparse_core_info()
NC, NS = _info.num_cores, _info.num_subcores
NW = NC * NS
R_PER_W = R // NW
IDX_W = 128
K = 5
CHUNK = IDX_W * K
N_CHUNKS = R_PER_W // CHUNK
IDX_ROWS = R_PER_W // IDX_W

_sc_mesh = plsc.VectorSubcoreMesh(core_axis_name="c", subcore_axis_name="s")


@functools.partial(
    pl.kernel,
    mesh=_sc_mesh,
    out_type=jax.ShapeDtypeStruct((R, E), jnp.float32),
    scratch_types=[
        pltpu.VMEM((IDX_ROWS, IDX_W), jnp.int32),
        pltpu.VMEM((CHUNK, E), jnp.float32),
        pltpu.SemaphoreType.DMA,
    ],
    compiler_params=pltpu.CompilerParams(use_tc_tiling_on_sc=False),
)
def _sc_gather(table_hbm, idx_hbm, out_hbm, idx_v, rows_v, sem):
    wid = lax.axis_index("s") * NC + lax.axis_index("c")
    base = wid * R_PER_W
    pltpu.sync_copy(idx_hbm.at[wid], idx_v)

    def chunk_body(i, carry):
        copies = []
        for j in range(K):
            copies.append(
                pltpu.async_copy(
                    table_hbm.at[idx_v.at[i * K + j]],
                    rows_v.at[pl.ds(j * IDX_W, IDX_W)],
                    sem,
                )
            )
        for c in copies:
            c.wait()
        pltpu.sync_copy(rows_v, out_hbm.at[pl.ds(base + i * CHUNK, CHUNK)])
        return carry

    lax.fori_loop(0, N_CHUNKS, chunk_body, 0)


def _ln_body(g_ref, seg_ref, ps0_ref, dps_ref, gam_ref, bet_ref, out_ref):
    e = g_ref[...]
    segf = seg_ref[...].astype(jnp.float32)[..., None]
    ps = ps0_ref[...][None, :, :] + segf * dps_ref[...].reshape(1, 1, E)
    e = e + ps
    mean = jnp.mean(e, axis=-1, keepdims=True)
    d = e - mean
    var = jnp.mean(d * d, axis=-1, keepdims=True)
    normed = d * lax.rsqrt(var + 1e-5)
    out_ref[...] = normed * gam_ref[...].reshape(1, 1, E) + bet_ref[...].reshape(1, 1, E)


def _tc_layernorm(gathered3, seg, ps0, dps, gam, bet):
    BB = 16
    return pl.pallas_call(
        _ln_body,
        grid=(B // BB,),
        in_specs=[
            pl.BlockSpec((BB, S, E), lambda i: (i, 0, 0)),
            pl.BlockSpec((BB, S), lambda i: (i, 0)),
            pl.BlockSpec((S, E), lambda i: (0, 0)),
            pl.BlockSpec((1, E), lambda i: (0, 0)),
            pl.BlockSpec((1, E), lambda i: (0, 0)),
            pl.BlockSpec((1, E), lambda i: (0, 0)),
        ],
        out_specs=pl.BlockSpec((BB, S, E), lambda i: (i, 0, 0)),
        out_shape=jax.ShapeDtypeStruct((B, S, E), jnp.float32),
    )(gathered3, seg, ps0, dps, gam, bet)


def kernel(x, segment_ids, token_table, pos_table, seg_table, ln_gamma, ln_beta):
    idx = x.reshape(NW, IDX_ROWS, IDX_W).astype(jnp.int32)
    gathered = _sc_gather(token_table, idx)
    ps0 = pos_table[:S] + seg_table[0][None, :]
    dps = (seg_table[1] - seg_table[0]).reshape(1, E)
    out = _tc_layernorm(
        gathered.reshape(B, S, E),
        segment_ids.astype(jnp.int32),
        ps0,
        dps,
        ln_gamma.reshape(1, E),
        ln_beta.reshape(1, E),
    )
    return out


# speedup vs baseline: 1.9846x; 1.0053x over previous
import functools

import jax
import jax.numpy as jnp
from jax import lax
from jax.experimental import pallas as pl
from jax.experimental.pallas import tpu as pltpu
from jax.experimental.pallas import tpu_sc as plsc

B = 1024
S = 200
E = 64
R = B * S

_info = plsc.get_sparse_core_info()
NC, NS = _info.num_cores, _info.num_subcores
NW = NC * NS
R_PER_W = R // NW
IDX_W = 128
K = 5
CHUNK = IDX_W * K
N_CHUNKS = R_PER_W // CHUNK
IDX_ROWS = R_PER_W // IDX_W

_sc_mesh = plsc.VectorSubcoreMesh(core_axis_name="c", subcore_axis_name="s")


@functools.partial(
    pl.kernel,
    mesh=_sc_mesh,
    out_type=jax.ShapeDtypeStruct((R, 2 * E), jnp.float32),
    scratch_types=[
        pltpu.VMEM((IDX_ROWS, IDX_W), jnp.int32),
        pltpu.VMEM((IDX_ROWS, IDX_W), jnp.int32),
        pltpu.VMEM((CHUNK, E), jnp.float32),
        pltpu.VMEM((CHUNK, E), jnp.float32),
        pltpu.SemaphoreType.DMA,
    ],
    compiler_params=pltpu.CompilerParams(use_tc_tiling_on_sc=False),
)
def _sc_gather(table_hbm, ps_hbm, idx_hbm, cidx_hbm, out_hbm,
               idx_v, cidx_v, rows_v, ps_v, sem):
    wid = lax.axis_index("s") * NC + lax.axis_index("c")
    base = wid * R_PER_W
    pltpu.sync_copy(idx_hbm.at[wid], idx_v)
    pltpu.sync_copy(cidx_hbm.at[wid], cidx_v)

    def chunk_body(i, carry):
        copies = []
        for j in range(K):
            copies.append(
                pltpu.async_copy(
                    table_hbm.at[idx_v.at[i * K + j]],
                    rows_v.at[pl.ds(j * IDX_W, IDX_W)],
                    sem,
                )
            )
            copies.append(
                pltpu.async_copy(
                    ps_hbm.at[cidx_v.at[i * K + j]],
                    ps_v.at[pl.ds(j * IDX_W, IDX_W)],
                    sem,
                )
            )
        for c in copies:
            c.wait()
        pltpu.sync_copy(
            rows_v, out_hbm.at[pl.ds(base + i * CHUNK, CHUNK), pl.ds(0, E)]
        )
        pltpu.sync_copy(
            ps_v, out_hbm.at[pl.ds(base + i * CHUNK, CHUNK), pl.ds(E, E)]
        )
        return carry

    lax.fori_loop(0, N_CHUNKS, chunk_body, 0)


SB = 8
BB = 256


def _ln_body(g_ref, gam_ref, bet_ref, out_ref):
    gam = gam_ref[...].reshape(1, 1, E)
    bet = bet_ref[...].reshape(1, 1, E)
    w = g_ref[...]
    e = w[:, :, :E] + w[:, :, E:]
    mean = jnp.mean(e, axis=-1, keepdims=True)
    d = e - mean
    var = jnp.mean(d * d, axis=-1, keepdims=True)
    normed = d * lax.rsqrt(var + 1e-5)
    res = normed * gam + bet
    for k in range(SB):
        out_ref[k, :, :] = res[:, k, :].T


def _tc_layernorm(gwide, gam, bet):
    return pl.pallas_call(
        _ln_body,
        grid=(S // SB, B // BB),
        in_specs=[
            pl.BlockSpec((BB, SB, 2 * E), lambda i, b: (b, i, 0)),
            pl.BlockSpec((1, E), lambda i, b: (0, 0)),
            pl.BlockSpec((1, E), lambda i, b: (0, 0)),
        ],
        out_specs=pl.BlockSpec((SB, E, BB), lambda i, b: (i, 0, b)),
        out_shape=jax.ShapeDtypeStruct((S, E, B), jnp.float32),
    )(gwide, gam, bet)


def kernel(x, segment_ids, token_table, pos_table, seg_table, ln_gamma, ln_beta):
    idx = x.reshape(NW, IDX_ROWS, IDX_W).astype(jnp.int32)
    ps_all = (pos_table[None, :S, :] + seg_table[:, None, :]).reshape(2 * S, E)
    cidx = (segment_ids.astype(jnp.int32) * S
            + jnp.arange(S, dtype=jnp.int32)[None, :]).reshape(NW, IDX_ROWS, IDX_W)
    gathered = _sc_gather(token_table, ps_all, idx, cidx)
    out_t = _tc_layernorm(
        gathered.reshape(B, S, 2 * E),
        ln_gamma.reshape(1, E),
        ln_beta.reshape(1, E),
    )
    return jnp.transpose(out_t, (2, 0, 1))


# speedup vs baseline: 2.1900x; 1.1035x over previous
import functools

import jax
import jax.numpy as jnp
from jax import lax
from jax.experimental import pallas as pl
from jax.experimental.pallas import tpu as pltpu
from jax.experimental.pallas import tpu_sc as plsc

B = 1024
S = 200
E = 64
R = B * S

_info = plsc.get_sparse_core_info()
NC, NS = _info.num_cores, _info.num_subcores
NW = NC * NS
R_PER_W = R // NW
IDX_W = 128
K = 5
CHUNK = IDX_W * K
N_CHUNKS = R_PER_W // CHUNK
IDX_ROWS = R_PER_W // IDX_W

_sc_mesh = plsc.VectorSubcoreMesh(core_axis_name="c", subcore_axis_name="s")


@functools.partial(
    pl.kernel,
    mesh=_sc_mesh,
    out_type=jax.ShapeDtypeStruct((R, 2 * E), jnp.float32),
    scratch_types=[
        pltpu.VMEM((IDX_ROWS, IDX_W), jnp.int32),
        pltpu.VMEM((IDX_ROWS, IDX_W), jnp.int32),
        pltpu.VMEM((CHUNK, E), jnp.float32),
        pltpu.VMEM((CHUNK, E), jnp.float32),
        pltpu.SemaphoreType.DMA,
    ],
    compiler_params=pltpu.CompilerParams(use_tc_tiling_on_sc=False),
)
def _sc_gather(table_hbm, ps_hbm, idx_hbm, cidx_hbm, out_hbm,
               idx_v, cidx_v, rows_v, ps_v, sem):
    wid = lax.axis_index("s") * NC + lax.axis_index("c")
    base = wid * R_PER_W
    pltpu.sync_copy(idx_hbm.at[wid], idx_v)
    pltpu.sync_copy(cidx_hbm.at[wid], cidx_v)

    def chunk_body(i, carry):
        copies = []
        for j in range(K):
            copies.append(
                pltpu.async_copy(
                    table_hbm.at[idx_v.at[i * K + j]],
                    rows_v.at[pl.ds(j * IDX_W, IDX_W)],
                    sem,
                )
            )
            copies.append(
                pltpu.async_copy(
                    ps_hbm.at[cidx_v.at[i * K + j]],
                    ps_v.at[pl.ds(j * IDX_W, IDX_W)],
                    sem,
                )
            )
        for c in copies:
            c.wait()
        pltpu.sync_copy(
            rows_v, out_hbm.at[pl.ds(base + i * CHUNK, CHUNK), pl.ds(0, E)]
        )
        pltpu.sync_copy(
            ps_v, out_hbm.at[pl.ds(base + i * CHUNK, CHUNK), pl.ds(E, E)]
        )
        return carry

    lax.fori_loop(0, N_CHUNKS, chunk_body, 0)


SB = 8
BB = 512


def _ln_body(g_ref, gam_ref, bet_ref, out_ref):
    gam = gam_ref[...].reshape(1, 1, E)
    bet = bet_ref[...].reshape(1, 1, E)
    w = g_ref[...]
    e = w[:, :, :E] + w[:, :, E:]
    mean = jnp.mean(e, axis=-1, keepdims=True)
    d = e - mean
    var = jnp.mean(d * d, axis=-1, keepdims=True)
    normed = d * lax.rsqrt(var + 1e-5)
    res = normed * gam + bet
    for k in range(SB):
        out_ref[k, :, :] = res[:, k, :].T


def _tc_layernorm(gwide, gam, bet):
    return pl.pallas_call(
        _ln_body,
        grid=(S // SB, B // BB),
        in_specs=[
            pl.BlockSpec((BB, SB, 2 * E), lambda i, b: (b, i, 0)),
            pl.BlockSpec((1, E), lambda i, b: (0, 0)),
            pl.BlockSpec((1, E), lambda i, b: (0, 0)),
        ],
        out_specs=pl.BlockSpec((SB, E, BB), lambda i, b: (i, 0, b)),
        out_shape=jax.ShapeDtypeStruct((S, E, B), jnp.float32),
    )(gwide, gam, bet)


def kernel(x, segment_ids, token_table, pos_table, seg_table, ln_gamma, ln_beta):
    idx = x.reshape(NW, IDX_ROWS, IDX_W).astype(jnp.int32)
    ps_one = (pos_table[None, :S, :] + seg_table[:, None, :]).reshape(2 * S, E)
    ps_all = jnp.broadcast_to(ps_one[None], (NW, 2 * S, E)).reshape(NW * 2 * S, E)
    cidx = (segment_ids.astype(jnp.int32) * S
            + jnp.arange(S, dtype=jnp.int32)[None, :]).reshape(NW, IDX_ROWS, IDX_W)
    cidx = cidx + (jnp.arange(NW, dtype=jnp.int32) * (2 * S))[:, None, None]
    gathered = _sc_gather(token_table, ps_all, idx, cidx)
    out_t = _tc_layernorm(
        gathered.reshape(B, S, 2 * E),
        ln_gamma.reshape(1, E),
        ln_beta.reshape(1, E),
    )
    return jnp.transpose(out_t, (2, 0, 1))
